# baseline (speedup 1.0000x reference)
import jax
import jax.numpy as jnp
from jax.experimental import pallas as pl
from jax.experimental.pallas import tpu as pltpu


def _remix_dma_kernel(perm_ref, in_ref, out_ref, sem):
    bs = in_ref.shape[1]
    copies = []
    for b in range(bs):
        row = perm_ref[b]
        copies.append(
            pltpu.make_async_copy(
                in_ref.at[0, pl.ds(row, 1)],
                out_ref.at[0, pl.ds(b, 1)],
                sem,
            )
        )
    copies.append(
        pltpu.make_async_copy(in_ref.at[1], out_ref.at[1], sem)
    )
    for cp in copies:
        cp.start()
    for cp in copies:
        cp.wait()


def kernel(sources):
    s2, bs, c, t = sources.shape
    perm_key = jax.random.key(42)
    perm = jnp.argsort(jax.random.uniform(perm_key, (bs,))).astype(jnp.int32)

    return pl.pallas_call(
        _remix_dma_kernel,
        grid_spec=pltpu.PrefetchScalarGridSpec(
            num_scalar_prefetch=1,
            grid=(),
            in_specs=[pl.BlockSpec(memory_space=pl.ANY)],
            out_specs=pl.BlockSpec(memory_space=pl.ANY),
            scratch_shapes=[pltpu.SemaphoreType.DMA],
        ),
        out_shape=jax.ShapeDtypeStruct(sources.shape, sources.dtype),
    )(perm, sources)


# speedup vs baseline: 3.7674x; 3.7674x over previous
import jax
import jax.numpy as jnp
from jax.experimental import pallas as pl
from jax.experimental.pallas import tpu as pltpu


def _copy_kernel(perm_ref, in_ref, out_ref):
    out_ref[...] = in_ref[...]


def kernel(sources):
    s2, bs, c, t = sources.shape
    perm_key = jax.random.key(42)
    perm = jnp.argsort(jax.random.uniform(perm_key, (bs,))).astype(jnp.int32)

    nchunk = 5
    tc = t // nchunk
    grid = (s2, bs, nchunk)

    def in_index(s, b, k, perm_ref):
        row = jnp.where(s == 0, perm_ref[b], b)
        return (s, row, 0, k)

    def out_index(s, b, k, perm_ref):
        return (s, b, 0, k)

    return pl.pallas_call(
        _copy_kernel,
        grid_spec=pltpu.PrefetchScalarGridSpec(
            num_scalar_prefetch=1,
            grid=grid,
            in_specs=[pl.BlockSpec((1, 1, c, tc), in_index)],
            out_specs=pl.BlockSpec((1, 1, c, tc), out_index),
        ),
        out_shape=jax.ShapeDtypeStruct(sources.shape, sources.dtype),
    )(perm, sources)


# speedup vs baseline: 34.8178x; 9.2420x over previous
import jax
import jax.numpy as jnp
from jax.experimental import pallas as pl
from jax.experimental.pallas import tpu as pltpu

_R = 8


def _copy_kernel(perm_ref, *refs):
    in_refs = refs[:_R]
    out_ref = refs[_R]
    for j in range(_R):
        out_ref[0, j] = in_refs[j][0, 0]


def kernel(sources):
    s2, bs, c, t = sources.shape
    perm_key = jax.random.key(42)
    perm = jnp.argsort(jax.random.uniform(perm_key, (bs,))).astype(jnp.int32)

    nblk = bs // _R
    grid = (s2, nblk)

    def make_in_index(j):
        def in_index(s, k, perm_ref):
            b = k * _R + j
            row = jnp.where(s == 0, perm_ref[b], b)
            return (s, row, 0, 0)
        return in_index

    def out_index(s, k, perm_ref):
        return (s, k, 0, 0)

    return pl.pallas_call(
        _copy_kernel,
        grid_spec=pltpu.PrefetchScalarGridSpec(
            num_scalar_prefetch=1,
            grid=grid,
            in_specs=[pl.BlockSpec((1, 1, c, t), make_in_index(j))
                      for j in range(_R)],
            out_specs=pl.BlockSpec((1, _R, c, t), out_index),
        ),
        out_shape=jax.ShapeDtypeStruct(sources.shape, sources.dtype),
    )(perm, *([sources] * _R))


# speedup vs baseline: 36.8670x; 1.0589x over previous
import jax
import jax.numpy as jnp
from jax.experimental import pallas as pl
from jax.experimental.pallas import tpu as pltpu

_R = 16


def _copy_kernel(perm_ref, *refs):
    in_refs = refs[:_R]
    out_ref = refs[_R]
    for j in range(_R):
        out_ref[0, j] = in_refs[j][0, 0]


def kernel(sources):
    s2, bs, c, t = sources.shape
    perm_key = jax.random.key(42)
    perm = jnp.argsort(jax.random.uniform(perm_key, (bs,))).astype(jnp.int32)

    nblk = bs // _R
    grid = (s2, nblk)

    def make_in_index(j):
        def in_index(s, k, perm_ref):
            b = k * _R + j
            row = jnp.where(s == 0, perm_ref[b], b)
            return (s, row, 0, 0)
        return in_index

    def out_index(s, k, perm_ref):
        return (s, k, 0, 0)

    return pl.pallas_call(
        _copy_kernel,
        grid_spec=pltpu.PrefetchScalarGridSpec(
            num_scalar_prefetch=1,
            grid=grid,
            in_specs=[pl.BlockSpec((1, 1, c, t), make_in_index(j))
                      for j in range(_R)],
            out_specs=pl.BlockSpec((1, _R, c, t), out_index),
        ),
        out_shape=jax.ShapeDtypeStruct(sources.shape, sources.dtype),
    )(perm, *([sources] * _R))


# speedup vs baseline: 40.0885x; 1.0874x over previous
import jax
import jax.numpy as jnp
from jax.experimental import pallas as pl
from jax.experimental.pallas import tpu as pltpu

_R = 32


def _copy_kernel(perm_ref, *refs):
    in_refs = refs[:_R]
    out_ref = refs[_R]
    for j in range(_R):
        out_ref[0, j] = in_refs[j][0, 0]


def kernel(sources):
    s2, bs, c, t = sources.shape
    perm_key = jax.random.key(42)
    perm = jnp.argsort(jax.random.uniform(perm_key, (bs,))).astype(jnp.int32)

    nblk = bs // _R
    grid = (s2, nblk)

    def make_in_index(j):
        def in_index(s, k, perm_ref):
            b = k * _R + j
            row = jnp.where(s == 0, perm_ref[b], b)
            return (s, row, 0, 0)
        return in_index

    def out_index(s, k, perm_ref):
        return (s, k, 0, 0)

    return pl.pallas_call(
        _copy_kernel,
        grid_spec=pltpu.PrefetchScalarGridSpec(
            num_scalar_prefetch=1,
            grid=grid,
            in_specs=[pl.BlockSpec((1, 1, c, t), make_in_index(j))
                      for j in range(_R)],
            out_specs=pl.BlockSpec((1, _R, c, t), out_index),
        ),
        out_shape=jax.ShapeDtypeStruct(sources.shape, sources.dtype),
    )(perm, *([sources] * _R))


# speedup vs baseline: 40.4489x; 1.0090x over previous
import jax
import jax.numpy as jnp
from jax.experimental import pallas as pl
from jax.experimental.pallas import tpu as pltpu

_G = 4
_RG = 8


def _remix_kernel(perm_ref, in_hbm, out_hbm, nscr, cscr, rsem_n, rsem_c, wsem):
    nreads = []
    for g in range(_G):
        for j in range(_RG):
            b = g * _RG + j
            row = perm_ref[b]
            cp = pltpu.make_async_copy(
                in_hbm.at[0, pl.ds(row, 1)],
                nscr.at[pl.ds(b, 1)],
                rsem_n.at[g],
            )
            cp.start()
            nreads.append(cp)
    creads = []
    for g in range(_G):
        cp = pltpu.make_async_copy(
            in_hbm.at[1, pl.ds(g * _RG, _RG)],
            cscr.at[pl.ds(g * _RG, _RG)],
            rsem_c.at[g],
        )
        cp.start()
        creads.append(cp)

    writes = []
    for g in range(_G):
        for j in range(_RG):
            nreads[g * _RG + j].wait()
        cp = pltpu.make_async_copy(
            nscr.at[pl.ds(g * _RG, _RG)],
            out_hbm.at[0, pl.ds(g * _RG, _RG)],
            wsem,
        )
        cp.start()
        writes.append(cp)
    for g in range(_G):
        creads[g].wait()
        cp = pltpu.make_async_copy(
            cscr.at[pl.ds(g * _RG, _RG)],
            out_hbm.at[1, pl.ds(g * _RG, _RG)],
            wsem,
        )
        cp.start()
        writes.append(cp)
    for cp in writes:
        cp.wait()


def kernel(sources):
    s2, bs, c, t = sources.shape
    perm_key = jax.random.key(42)
    perm = jnp.argsort(jax.random.uniform(perm_key, (bs,))).astype(jnp.int32)

    return pl.pallas_call(
        _remix_kernel,
        grid_spec=pltpu.PrefetchScalarGridSpec(
            num_scalar_prefetch=1,
            grid=(),
            in_specs=[pl.BlockSpec(memory_space=pl.ANY)],
            out_specs=pl.BlockSpec(memory_space=pl.ANY),
            scratch_shapes=[
                pltpu.VMEM((bs, c, t), sources.dtype),
                pltpu.VMEM((bs, c, t), sources.dtype),
                pltpu.SemaphoreType.DMA((_G,)),
                pltpu.SemaphoreType.DMA((_G,)),
                pltpu.SemaphoreType.DMA,
            ],
        ),
        out_shape=jax.ShapeDtypeStruct(sources.shape, sources.dtype),
    )(perm, sources)
